# baseline (speedup 1.0000x reference)
import jax
import jax.numpy as jnp
from jax import lax
from jax.experimental import pallas as pl
from jax.experimental.pallas import tpu as pltpu
from jax.experimental.pallas import tpu_sc as plsc

B = 16
N = 4096
C = 64
T = N // 2
R = 1024
UNM = T - R
TT = 256
NT = T // TT
OUT_ROWS = UNM + T
COMB_ROWS = OUT_ROWS + 8


def _tc_body(pts_ref, met_ref, srcs_ref, dsts_ref, cidx_ref):
    src = pts_ref[0, :, 0:64]
    dst = pts_ref[0, :, 64:128]
    b_n = met_ref[0, :, 64:128]

    ones_col = jnp.ones((T, 1), jnp.float32)

    vmax_parts = []
    nidx_parts = []
    for tt in range(NT):
        a_n = met_ref[0, tt * TT:(tt + 1) * TT, 0:64]
        raw = jax.lax.dot_general(
            a_n, b_n, (((1,), (1,)), ((), ())),
            preferred_element_type=jnp.float32)
        m = jnp.max(raw, axis=1)
        nidx = jnp.argmax(raw, axis=1)
        vmax_parts.append(m)
        nidx_parts.append(nidx.astype(jnp.int32))

    vrow = jnp.concatenate([p[None, :] for p in vmax_parts], axis=1)
    jrow = jax.lax.broadcasted_iota(jnp.int32, (1, T), 1)

    rank_parts = []
    for tt in range(NT):
        vcol = vmax_parts[tt][:, None]
        icol = (jax.lax.broadcasted_iota(jnp.int32, (TT, 1), 0)
                + tt * TT)
        pred = (vrow > vcol) | ((vrow == vcol) & (jrow < icol))
        rank_f = jax.lax.dot_general(
            pred.astype(jnp.float32), ones_col, (((1,), (0,)), ((), ())),
            preferred_element_type=jnp.float32)
        rank_parts.append(rank_f[:, 0].astype(jnp.int32))

    cidx_parts = []
    for tt in range(NT):
        keep = rank_parts[tt] < R
        cidx_parts.append(
            jnp.where(keep, UNM + nidx_parts[tt], rank_parts[tt] - R))
    cidx_row = jnp.concatenate([p[None, :] for p in cidx_parts], axis=1)

    invc_parts = []
    for tt in range(NT):
        dcol = (jax.lax.broadcasted_iota(jnp.int32, (TT, 1), 0)
                + tt * TT + UNM)
        eq_f = (cidx_row == dcol).astype(jnp.float32)
        cnt = 1.0 + jax.lax.dot_general(
            eq_f, ones_col, (((1,), (0,)), ((), ())),
            preferred_element_type=jnp.float32)[:, 0]
        invc_parts.append(1.0 / cnt)
    invc_row = jnp.concatenate([p[None, :] for p in invc_parts], axis=1)

    zpad = jnp.zeros((TT, C), jnp.float32)
    for tt in range(NT):
        nidx_col = nidx_parts[tt][:, None]
        w = jnp.sum(jnp.where(jrow == nidx_col, invc_row, 0.0), axis=1)
        keep = rank_parts[tt] < R
        w = jnp.where(keep, w, 1.0)
        srcs_ref[0, tt * TT:(tt + 1) * TT, :] = jnp.concatenate(
            [src[tt * TT:(tt + 1) * TT, :] * w[:, None], zpad], axis=1)
        dsts_ref[0, tt * TT:(tt + 1) * TT, :] = jnp.concatenate(
            [dst[tt * TT:(tt + 1) * TT, :] * invc_parts[tt][:, None], zpad],
            axis=1)
        cidx_ref[0, tt * 2:(tt + 1) * 2, :] = (
            cidx_parts[tt].astype(jnp.int32).reshape(2, 128))


def _tc_stage(pts2, met2, interpret=False):
    nb = pts2.shape[0]
    return pl.pallas_call(
        _tc_body,
        grid=(nb,),
        in_specs=[pl.BlockSpec((1, T, 2 * C), lambda b: (b, 0, 0)),
                  pl.BlockSpec((1, T, 2 * C), lambda b: (b, 0, 0))],
        out_specs=[
            pl.BlockSpec((1, T, 2 * C), lambda b: (b, 0, 0)),
            pl.BlockSpec((1, T, 2 * C), lambda b: (b, 0, 0)),
            pl.BlockSpec((1, 16, 128), lambda b: (b, 0, 0)),
        ],
        out_shape=[
            jax.ShapeDtypeStruct((nb, T, 2 * C), jnp.float32),
            jax.ShapeDtypeStruct((nb, T, 2 * C), jnp.float32),
            jax.ShapeDtypeStruct((nb, 16, 128), jnp.int32),
        ],
        interpret=interpret,
    )(pts2, met2)


Q = 4
W = T // Q
UQ = UNM // Q
OQ = OUT_ROWS // Q


def _sc_body(srcs_hbm, dsts_hbm, cidx_hbm, zero_hbm, out_hbm,
             cidx_v, rows_v, comb_sh):
    c = lax.axis_index("c")
    s = lax.axis_index("s")
    bb = s // Q
    q = s % Q
    b = c * 4 + bb

    pltpu.sync_copy(zero_hbm, comb_sh.at[bb, pl.ds(q * UQ, UQ)])
    pltpu.sync_copy(dsts_hbm.at[b, pl.ds(q * W, W)],
                    comb_sh.at[bb, pl.ds(UNM + q * W, W)])
    pltpu.sync_copy(cidx_hbm.at[b, pl.ds(q * 4, 4)], cidx_v)
    plsc.subcore_barrier()

    for ch in range(4):
        pltpu.sync_copy(srcs_hbm.at[b, pl.ds(q * W + ch * 128, 128)],
                        rows_v)
        pltpu.sync_copy(rows_v, comb_sh.at[bb].at[cidx_v.at[ch]],
                        add=True)
    plsc.subcore_barrier()

    pltpu.sync_copy(comb_sh.at[bb, pl.ds(q * OQ, OQ)],
                    out_hbm.at[b, pl.ds(q * OQ, OQ)])


def _sc_stage(srcs, dsts, cidx, zeros):
    nb = srcs.shape[0]
    mesh = plsc.VectorSubcoreMesh(core_axis_name="c", subcore_axis_name="s")
    return pl.kernel(
        _sc_body,
        out_type=jax.ShapeDtypeStruct((nb, OUT_ROWS, 2 * C), jnp.float32),
        mesh=mesh,
        scratch_types=[
            pltpu.VMEM((4, 128), jnp.int32),
            pltpu.VMEM((128, 2 * C), jnp.float32),
            pltpu.VMEM_SHARED((4, COMB_ROWS, 2 * C), jnp.float32),
        ],
        compiler_params=pltpu.CompilerParams(use_tc_tiling_on_sc=True),
    )(srcs, dsts, cidx, zeros)


@jax.jit
def kernel(points):
    zeros = jnp.zeros((UQ, 2 * C), jnp.float32)
    halves = []
    for h in range(2):
        ptsh = points[h * (B // 2):(h + 1) * (B // 2)]
        metric = ptsh / jnp.linalg.norm(ptsh, axis=-1, keepdims=True)
        pts2 = ptsh.reshape(B // 2, T, 2 * C)
        met2 = metric.reshape(B // 2, T, 2 * C)
        halves.append(_tc_stage(pts2, met2))
    outs = [_sc_stage(*halves[h], zeros) for h in range(2)]
    return jnp.concatenate(outs, axis=0)[:, :, :C]


# speedup vs baseline: 1.1045x; 1.1045x over previous
import jax
import jax.numpy as jnp
from jax import lax
from jax.experimental import pallas as pl
from jax.experimental.pallas import tpu as pltpu
from jax.experimental.pallas import tpu_sc as plsc

B = 16
N = 4096
C = 64
T = N // 2
R = 1024
UNM = T - R
TT = 256
NT = T // TT
OUT_ROWS = UNM + T
COMB_ROWS = OUT_ROWS + 8


def _tc_body(pts_ref, met_ref, srcs_ref, dsts_ref, cidx_ref):
    src = pts_ref[0, :, 0:64]
    dst = pts_ref[0, :, 64:128]
    b_n = met_ref[0, :, 64:128]

    ones_col = jnp.ones((T, 1), jnp.float32)

    vmax_parts = []
    nidx_parts = []
    for tt in range(NT):
        a_n = met_ref[0, tt * TT:(tt + 1) * TT, 0:64]
        raw = jax.lax.dot_general(
            a_n, b_n, (((1,), (1,)), ((), ())),
            preferred_element_type=jnp.float32)
        m = jnp.max(raw, axis=1)
        nidx = jnp.argmax(raw, axis=1)
        vmax_parts.append(m)
        nidx_parts.append(nidx.astype(jnp.int32))

    vrow = jnp.concatenate([p[None, :] for p in vmax_parts], axis=1)
    jrow = jax.lax.broadcasted_iota(jnp.int32, (1, T), 1)

    rank_parts = []
    for tt in range(NT):
        vcol = vmax_parts[tt][:, None]
        icol = (jax.lax.broadcasted_iota(jnp.int32, (TT, 1), 0)
                + tt * TT)
        pred = (vrow > vcol) | ((vrow == vcol) & (jrow < icol))
        rank_f = jax.lax.dot_general(
            pred.astype(jnp.float32), ones_col, (((1,), (0,)), ((), ())),
            preferred_element_type=jnp.float32)
        rank_parts.append(rank_f[:, 0].astype(jnp.int32))

    cidx_parts = []
    for tt in range(NT):
        keep = rank_parts[tt] < R
        cidx_parts.append(
            jnp.where(keep, UNM + nidx_parts[tt], rank_parts[tt] - R))
    cidx_row = jnp.concatenate([p[None, :] for p in cidx_parts], axis=1)

    invc_parts = []
    for tt in range(NT):
        dcol = (jax.lax.broadcasted_iota(jnp.int32, (TT, 1), 0)
                + tt * TT + UNM)
        eq_f = (cidx_row == dcol).astype(jnp.float32)
        cnt = 1.0 + jax.lax.dot_general(
            eq_f, ones_col, (((1,), (0,)), ((), ())),
            preferred_element_type=jnp.float32)[:, 0]
        invc_parts.append(1.0 / cnt)
    invc_row = jnp.concatenate([p[None, :] for p in invc_parts], axis=1)

    zpad = jnp.zeros((TT, C), jnp.float32)
    for tt in range(NT):
        nidx_col = nidx_parts[tt][:, None]
        w = jnp.sum(jnp.where(jrow == nidx_col, invc_row, 0.0), axis=1)
        keep = rank_parts[tt] < R
        w = jnp.where(keep, w, 1.0)
        srcs_ref[0, tt * TT:(tt + 1) * TT, :] = jnp.concatenate(
            [src[tt * TT:(tt + 1) * TT, :] * w[:, None], zpad], axis=1)
        dsts_ref[0, tt * TT:(tt + 1) * TT, :] = jnp.concatenate(
            [dst[tt * TT:(tt + 1) * TT, :] * invc_parts[tt][:, None], zpad],
            axis=1)
        cidx_ref[0, tt * 2:(tt + 1) * 2, :] = (
            cidx_parts[tt].astype(jnp.int32).reshape(2, 128))


def _tc_stage(pts2, met2, interpret=False):
    nb = pts2.shape[0]
    return pl.pallas_call(
        _tc_body,
        grid=(nb,),
        in_specs=[pl.BlockSpec((1, T, 2 * C), lambda b: (b, 0, 0)),
                  pl.BlockSpec((1, T, 2 * C), lambda b: (b, 0, 0))],
        out_specs=[
            pl.BlockSpec((1, T, 2 * C), lambda b: (b, 0, 0)),
            pl.BlockSpec((1, T, 2 * C), lambda b: (b, 0, 0)),
            pl.BlockSpec((1, 16, 128), lambda b: (b, 0, 0)),
        ],
        out_shape=[
            jax.ShapeDtypeStruct((nb, T, 2 * C), jnp.float32),
            jax.ShapeDtypeStruct((nb, T, 2 * C), jnp.float32),
            jax.ShapeDtypeStruct((nb, 16, 128), jnp.int32),
        ],
        interpret=interpret,
    )(pts2, met2)


Q = 4
W = T // Q
UQ = UNM // Q
OQ = OUT_ROWS // Q


def _sc_body(srcs_hbm, dsts_hbm, cidx_hbm, zero_hbm, out_hbm,
             cidx_v, rows_v, comb_sh):
    c = lax.axis_index("c")
    s = lax.axis_index("s")
    bb = s // Q
    q = s % Q

    for rd in range(2):
        b = c * 8 + rd * 4 + bb

        pltpu.sync_copy(zero_hbm, comb_sh.at[bb, pl.ds(q * UQ, UQ)])
        pltpu.sync_copy(dsts_hbm.at[b, pl.ds(q * W, W)],
                        comb_sh.at[bb, pl.ds(UNM + q * W, W)])
        pltpu.sync_copy(cidx_hbm.at[b, pl.ds(q * 4, 4)], cidx_v)
        plsc.subcore_barrier()

        for ch in range(4):
            pltpu.sync_copy(srcs_hbm.at[b, pl.ds(q * W + ch * 128, 128)],
                            rows_v)
            pltpu.sync_copy(rows_v, comb_sh.at[bb].at[cidx_v.at[ch]],
                            add=True)
        plsc.subcore_barrier()

        pltpu.sync_copy(comb_sh.at[bb, pl.ds(q * OQ, OQ)],
                        out_hbm.at[b, pl.ds(q * OQ, OQ)])
        if rd == 0:
            plsc.subcore_barrier()


def _sc_stage(srcs, dsts, cidx, zeros):
    nb = srcs.shape[0]
    mesh = plsc.VectorSubcoreMesh(core_axis_name="c", subcore_axis_name="s")
    return pl.kernel(
        _sc_body,
        out_type=jax.ShapeDtypeStruct((nb, OUT_ROWS, 2 * C), jnp.float32),
        mesh=mesh,
        scratch_types=[
            pltpu.VMEM((4, 128), jnp.int32),
            pltpu.VMEM((128, 2 * C), jnp.float32),
            pltpu.VMEM_SHARED((4, COMB_ROWS, 2 * C), jnp.float32),
        ],
        compiler_params=pltpu.CompilerParams(use_tc_tiling_on_sc=True),
    )(srcs, dsts, cidx, zeros)


@jax.jit
def kernel(points):
    metric = points / jnp.linalg.norm(points, axis=-1, keepdims=True)
    pts2 = points.reshape(B, T, 2 * C)
    met2 = metric.reshape(B, T, 2 * C)
    srcs, dsts, cidx = _tc_stage(pts2, met2)
    zeros = jnp.zeros((UQ, 2 * C), jnp.float32)
    out2 = _sc_stage(srcs, dsts, cidx, zeros)
    return out2[:, :, :C]
